# baseline (speedup 1.0000x reference)
import functools
import math

import jax
import jax.numpy as jnp
from jax import lax
from jax.experimental import pallas as pl
from jax.experimental.pallas import tpu as pltpu
from jax.experimental.pallas import tpu_sc as plsc

NUM_CORES = 2
NUM_SUBCORES = 16
NUM_WORKERS = NUM_CORES * NUM_SUBCORES
LANES = 16
CHUNK = 128


@functools.cache
def _build_transpose(n_rows, dim):
    blocks = n_rows // CHUNK
    out_rows = n_rows * dim // 128
    tail_out = out_rows - blocks * dim
    per_w = math.ceil(blocks / NUM_WORKERS)
    mesh = plsc.VectorSubcoreMesh(
        core_axis_name="c", subcore_axis_name="s",
        num_cores=NUM_CORES, num_subcores=NUM_SUBCORES)

    @functools.partial(
        pl.kernel,
        out_type=jax.ShapeDtypeStruct((out_rows, 128), jnp.float32),
        mesh=mesh,
        scratch_types=[
            pltpu.VMEM((2, dim, CHUNK), jnp.float32),
            pltpu.VMEM((2, dim, CHUNK), jnp.float32),
            pltpu.VMEM((tail_out, 128), jnp.float32),
            pltpu.SemaphoreType.DMA,
            pltpu.SemaphoreType.DMA,
            pltpu.SemaphoreType.DMA,
            pltpu.SemaphoreType.DMA,
        ],
        compiler_params=pltpu.CompilerParams(
            use_tc_tiling_on_sc=True, needs_layout_passes=False),
    )
    def tkern(tab_t_hbm, tail_hbm, out_hbm, inb, outb, tailv,
              isem0, isem1, osem0, osem1):
        wid = lax.axis_index("s") * NUM_CORES + lax.axis_index("c")
        start = wid * blocks // NUM_WORKERS
        isems = (isem0, isem1)
        osems = (osem0, osem1)
        iota = lax.iota(jnp.int32, LANES)

        def in_sl(j):
            off = pl.multiple_of((start + j) * CHUNK, CHUNK)
            return tab_t_hbm.at[:, pl.ds(off, CHUNK)]

        def out_sl(j):
            off = pl.multiple_of((start + j) * dim, dim)
            return out_hbm.at[pl.ds(off, dim)]

        def fire_in(j, p):
            pltpu.async_copy(in_sl(j), inb.at[p], isems[p])

        def wait_in(j, p):
            pltpu.make_async_copy(in_sl(j), inb.at[p], isems[p]).wait()

        def fire_out(j, p):
            pltpu.async_copy(outb.at[p], out_sl(j), osems[p])

        def wait_out(j, p):
            pltpu.make_async_copy(outb.at[p], out_sl(j), osems[p]).wait()

        def transpose_block(p):
            def rbody(rr, _):
                for k in range(8):
                    r = rr * 8 + k
                    va = plsc.load_gather(inb.at[p], [iota, iota * 0 + r])
                    outb[p, rr, pl.ds(k * dim, dim)] = va
                return 0

            lax.fori_loop(0, CHUNK // 8, rbody, 0)

        def step(j, p, prefetch_guard):
            wait_in(j, p)
            transpose_block(p)
            fire_out(j, p)
            if prefetch_guard is None:
                fire_in(j + 2, p)
            elif prefetch_guard:
                @pl.when(j + 2 < per_w)
                def _():
                    fire_in(j + 2, p)

        fire_in(0, 0)
        fire_in(1, 1)
        step(0, 0, None)
        step(1, 1, None)

        def jbody(jj, _):
            for sub in range(2):
                j = jj * 2 + sub
                p = sub
                wait_in(j, p)
                wait_out(j - 2, p)
                transpose_block(p)
                fire_out(j, p)

                @pl.when(j + 2 < per_w)
                def _():
                    fire_in(j + 2, p)
            return 0

        half = per_w // 2
        lax.fori_loop(1, half, jbody, 0)
        j_last = per_w - 1
        wait_in(j_last, 0)
        wait_out(j_last - 2, 0)
        transpose_block(0)
        fire_out(j_last, 0)
        wait_out(j_last - 1, 1)
        wait_out(j_last, 0)

        @pl.when(wid == 0)
        def _():
            pltpu.sync_copy(tail_hbm, tailv)
            pltpu.sync_copy(tailv, out_hbm.at[pl.ds(blocks * dim, tail_out)])

    return tkern, blocks


@functools.cache
def _build_gather(batch, table_slices, dim):
    lp = 128 // dim
    lp_shift = lp.bit_length() - 1
    b_per_w = batch // NUM_WORKERS
    n_chunks = b_per_w // CHUNK
    gpc = CHUNK // LANES
    out_rows_w = b_per_w * dim // 128
    mesh = plsc.VectorSubcoreMesh(
        core_axis_name="c", subcore_axis_name="s",
        num_cores=NUM_CORES, num_subcores=NUM_SUBCORES)

    @functools.partial(
        pl.kernel,
        out_type=jax.ShapeDtypeStruct((batch * dim // 128, 128), jnp.float32),
        mesh=mesh,
        scratch_types=[
            pltpu.VMEM((b_per_w,), jnp.int32),
            pltpu.VMEM((b_per_w,), jnp.int32),
            pltpu.VMEM((n_chunks, CHUNK), jnp.int32),
            pltpu.VMEM((n_chunks, CHUNK), jnp.int32),
            pltpu.VMEM((2, CHUNK, 128), jnp.float32),
            pltpu.VMEM((2, CHUNK, 128), jnp.float32),
            pltpu.VMEM((out_rows_w, 128), jnp.float32),
            pltpu.SemaphoreType.DMA,
            pltpu.SemaphoreType.DMA,
        ],
        compiler_params=pltpu.CompilerParams(
            use_tc_tiling_on_sc=True, needs_layout_passes=False),
    )
    def gmf(idx_a_hbm, idx_b_hbm, table_hbm, out_hbm,
            idx_av, idx_bv, qa_v, qb_v, buf_a, buf_b, out_v, sem0, sem1):
        wid = lax.axis_index("s") * NUM_CORES + lax.axis_index("c")
        base = wid * b_per_w
        pltpu.sync_copy(idx_a_hbm.at[pl.ds(base, b_per_w)], idx_av)
        pltpu.sync_copy(idx_b_hbm.at[pl.ds(base, b_per_w)], idx_bv)

        for k in range(b_per_w // LANES):
            sl = pl.ds(k * LANES, LANES)
            dst = pl.ds((k % gpc) * LANES, LANES)
            qa_v[k // gpc, dst] = lax.shift_right_logical(idx_av[sl], lp_shift)
            qb_v[k // gpc, dst] = lax.shift_right_logical(idx_bv[sl], lp_shift)

        sems = (sem0, sem1)

        def fire(j):
            p = j % 2
            return (
                pltpu.async_copy(table_hbm.at[qa_v.at[j]], buf_a.at[p], sems[p]),
                pltpu.async_copy(table_hbm.at[qb_v.at[j]], buf_b.at[p], sems[p]),
            )

        iota = lax.iota(jnp.int32, LANES)
        ocol = jnp.bitwise_and(iota, lp - 1) * dim
        orow_l = lax.shift_right_logical(iota, lp_shift)

        pending = {0: fire(0)}
        if n_chunks > 1:
            pending[1] = fire(1)
        for j in range(n_chunks):
            p = j % 2
            da, db = pending.pop(j)
            da.wait()
            db.wait()
            for g in range(gpc):
                rows_i = iota + g * LANES
                sl = pl.ds(j * CHUNK + g * LANES, LANES)
                cola = jnp.bitwise_and(idx_av[sl], lp - 1) * dim
                colb = jnp.bitwise_and(idx_bv[sl], lp - 1) * dim
                orow = orow_l + (j * CHUNK + g * LANES) // lp

                def dbody(d, _, rows_i=rows_i, cola=cola, colb=colb,
                          orow=orow, p=p):
                    va = plsc.load_gather(buf_a.at[p], [rows_i, cola + d])
                    vb = plsc.load_gather(buf_b.at[p], [rows_i, colb + d])
                    plsc.store_scatter(out_v, [orow, ocol + d], va * vb)
                    return 0

                lax.fori_loop(0, dim, dbody, 0, unroll=4)
            if j + 2 < n_chunks:
                pending[j + 2] = fire(j + 2)

        pltpu.sync_copy(out_v, out_hbm.at[pl.ds(wid * out_rows_w, out_rows_w)])

    return gmf


def kernel(input_plylst, input_item, table_plylst, table_item):
    batch = input_plylst.shape[0]
    n_rows, dim = table_plylst.shape
    idx_a = input_plylst.astype(jnp.int32)
    idx_b = input_item.astype(jnp.int32)
    tkern, blocks = _build_transpose(n_rows, dim)
    tail = table_plylst[blocks * CHUNK:].reshape(-1, 128)
    table128 = tkern(table_plylst.T, tail)
    out = _build_gather(batch, table128.shape[0], dim)(idx_a, idx_b, table128)
    return out.reshape(batch, dim)


# speedup vs baseline: 1.6438x; 1.6438x over previous
import functools
import math

import jax
import jax.numpy as jnp
from jax import lax
from jax.experimental import pallas as pl
from jax.experimental.pallas import tpu as pltpu
from jax.experimental.pallas import tpu_sc as plsc

NUM_CORES = 2
NUM_SUBCORES = 16
NUM_WORKERS = NUM_CORES * NUM_SUBCORES
LANES = 16
CHUNK = 128


@functools.cache
def _build_transpose(n_rows, dim):
    blocks = n_rows // CHUNK
    out_rows = n_rows * dim // 128
    tail_out = out_rows - blocks * dim
    per_w = math.ceil(blocks / NUM_WORKERS)
    mesh = plsc.VectorSubcoreMesh(
        core_axis_name="c", subcore_axis_name="s",
        num_cores=NUM_CORES, num_subcores=NUM_SUBCORES)

    @functools.partial(
        pl.kernel,
        out_type=jax.ShapeDtypeStruct((out_rows, 128), jnp.float32),
        mesh=mesh,
        scratch_types=[
            pltpu.VMEM((2, dim, CHUNK), jnp.float32),
            pltpu.VMEM((2, dim, CHUNK), jnp.float32),
            pltpu.VMEM((tail_out, 128), jnp.float32),
            pltpu.SemaphoreType.DMA,
            pltpu.SemaphoreType.DMA,
            pltpu.SemaphoreType.DMA,
            pltpu.SemaphoreType.DMA,
        ],
        compiler_params=pltpu.CompilerParams(
            use_tc_tiling_on_sc=True, needs_layout_passes=False),
    )
    def tkern(tab_t_hbm, tail_hbm, out_hbm, inb, outb, tailv,
              isem0, isem1, osem0, osem1):
        wid = lax.axis_index("s") * NUM_CORES + lax.axis_index("c")
        start = wid * blocks // NUM_WORKERS
        isems = (isem0, isem1)
        osems = (osem0, osem1)
        iota = lax.iota(jnp.int32, LANES)

        def in_sl(j):
            off = pl.multiple_of((start + j) * CHUNK, CHUNK)
            return tab_t_hbm.at[:, pl.ds(off, CHUNK)]

        def out_sl(j):
            off = pl.multiple_of((start + j) * dim, dim)
            return out_hbm.at[pl.ds(off, dim)]

        def fire_in(j, p):
            pltpu.async_copy(in_sl(j), inb.at[p], isems[p])

        def wait_in(j, p):
            pltpu.make_async_copy(in_sl(j), inb.at[p], isems[p]).wait()

        def fire_out(j, p):
            pltpu.async_copy(outb.at[p], out_sl(j), osems[p])

        def wait_out(j, p):
            pltpu.make_async_copy(outb.at[p], out_sl(j), osems[p]).wait()

        def transpose_block(p):
            @plsc.parallel_loop(0, CHUNK // 8, unroll=2)
            def rbody(rr):
                for k in range(8):
                    r = rr * 8 + k
                    va = plsc.load_gather(inb.at[p], [iota, iota * 0 + r])
                    outb[p, rr, pl.ds(k * dim, dim)] = va

        def step(j, p, prefetch_guard):
            wait_in(j, p)
            transpose_block(p)
            fire_out(j, p)
            if prefetch_guard is None:
                fire_in(j + 2, p)
            elif prefetch_guard:
                @pl.when(j + 2 < per_w)
                def _():
                    fire_in(j + 2, p)

        fire_in(0, 0)
        fire_in(1, 1)
        step(0, 0, None)
        step(1, 1, None)

        def jbody(jj, _):
            for sub in range(2):
                j = jj * 2 + sub
                p = sub
                wait_in(j, p)
                wait_out(j - 2, p)
                transpose_block(p)
                fire_out(j, p)

                @pl.when(j + 2 < per_w)
                def _():
                    fire_in(j + 2, p)
            return 0

        half = per_w // 2
        lax.fori_loop(1, half, jbody, 0)
        j_last = per_w - 1
        wait_in(j_last, 0)
        wait_out(j_last - 2, 0)
        transpose_block(0)
        fire_out(j_last, 0)
        wait_out(j_last - 1, 1)
        wait_out(j_last, 0)

        @pl.when(wid == 0)
        def _():
            pltpu.sync_copy(tail_hbm, tailv)
            pltpu.sync_copy(tailv, out_hbm.at[pl.ds(blocks * dim, tail_out)])

    return tkern, blocks


@functools.cache
def _build_gather(batch, table_slices, dim):
    lp = 128 // dim
    lp_shift = lp.bit_length() - 1
    b_per_w = batch // NUM_WORKERS
    n_chunks = b_per_w // CHUNK
    gpc = CHUNK // LANES
    out_rows_w = b_per_w * dim // 128
    mesh = plsc.VectorSubcoreMesh(
        core_axis_name="c", subcore_axis_name="s",
        num_cores=NUM_CORES, num_subcores=NUM_SUBCORES)

    @functools.partial(
        pl.kernel,
        out_type=jax.ShapeDtypeStruct((batch * dim // 128, 128), jnp.float32),
        mesh=mesh,
        scratch_types=[
            pltpu.VMEM((b_per_w,), jnp.int32),
            pltpu.VMEM((b_per_w,), jnp.int32),
            pltpu.VMEM((n_chunks, CHUNK), jnp.int32),
            pltpu.VMEM((n_chunks, CHUNK), jnp.int32),
            pltpu.VMEM((2, CHUNK, 128), jnp.float32),
            pltpu.VMEM((2, CHUNK, 128), jnp.float32),
            pltpu.VMEM((out_rows_w, 128), jnp.float32),
            pltpu.SemaphoreType.DMA,
            pltpu.SemaphoreType.DMA,
        ],
        compiler_params=pltpu.CompilerParams(
            use_tc_tiling_on_sc=True, needs_layout_passes=False),
    )
    def gmf(idx_a_hbm, idx_b_hbm, table_hbm, out_hbm,
            idx_av, idx_bv, qa_v, qb_v, buf_a, buf_b, out_v, sem0, sem1):
        wid = lax.axis_index("s") * NUM_CORES + lax.axis_index("c")
        base = wid * b_per_w
        pltpu.sync_copy(idx_a_hbm.at[pl.ds(base, b_per_w)], idx_av)
        pltpu.sync_copy(idx_b_hbm.at[pl.ds(base, b_per_w)], idx_bv)

        for k in range(b_per_w // LANES):
            sl = pl.ds(k * LANES, LANES)
            dst = pl.ds((k % gpc) * LANES, LANES)
            qa_v[k // gpc, dst] = lax.shift_right_logical(idx_av[sl], lp_shift)
            qb_v[k // gpc, dst] = lax.shift_right_logical(idx_bv[sl], lp_shift)

        sems = (sem0, sem1)

        def fire(j):
            p = j % 2
            return (
                pltpu.async_copy(table_hbm.at[qa_v.at[j]], buf_a.at[p], sems[p]),
                pltpu.async_copy(table_hbm.at[qb_v.at[j]], buf_b.at[p], sems[p]),
            )

        iota = lax.iota(jnp.int32, LANES)
        ocol = jnp.bitwise_and(iota, lp - 1) * dim
        orow_l = lax.shift_right_logical(iota, lp_shift)

        pending = {0: fire(0)}
        if n_chunks > 1:
            pending[1] = fire(1)
        for j in range(n_chunks):
            p = j % 2
            da, db = pending.pop(j)
            da.wait()
            db.wait()
            for g in range(gpc):
                rows_i = iota + g * LANES
                sl = pl.ds(j * CHUNK + g * LANES, LANES)
                cola = jnp.bitwise_and(idx_av[sl], lp - 1) * dim
                colb = jnp.bitwise_and(idx_bv[sl], lp - 1) * dim
                orow = orow_l + (j * CHUNK + g * LANES) // lp

                def dbody(d, _, rows_i=rows_i, cola=cola, colb=colb,
                          orow=orow, p=p):
                    va = plsc.load_gather(buf_a.at[p], [rows_i, cola + d])
                    vb = plsc.load_gather(buf_b.at[p], [rows_i, colb + d])
                    plsc.store_scatter(out_v, [orow, ocol + d], va * vb)
                    return 0

                lax.fori_loop(0, dim, dbody, 0, unroll=4)
            if j + 2 < n_chunks:
                pending[j + 2] = fire(j + 2)

        pltpu.sync_copy(out_v, out_hbm.at[pl.ds(wid * out_rows_w, out_rows_w)])

    return gmf


def kernel(input_plylst, input_item, table_plylst, table_item):
    batch = input_plylst.shape[0]
    n_rows, dim = table_plylst.shape
    idx_a = input_plylst.astype(jnp.int32)
    idx_b = input_item.astype(jnp.int32)
    tkern, blocks = _build_transpose(n_rows, dim)
    tail = table_plylst[blocks * CHUNK:].reshape(-1, 128)
    table128 = tkern(table_plylst.T, tail)
    out = _build_gather(batch, table128.shape[0], dim)(idx_a, idx_b, table128)
    return out.reshape(batch, dim)


# speedup vs baseline: 1.6478x; 1.0025x over previous
import functools
import math

import jax
import jax.numpy as jnp
from jax import lax
from jax.experimental import pallas as pl
from jax.experimental.pallas import tpu as pltpu
from jax.experimental.pallas import tpu_sc as plsc

NUM_CORES = 2
NUM_SUBCORES = 16
NUM_WORKERS = NUM_CORES * NUM_SUBCORES
LANES = 16
CHUNK = 128


@functools.cache
def _build_transpose(n_rows, dim):
    blocks = n_rows // CHUNK
    out_rows = n_rows * dim // 128
    tail_out = out_rows - blocks * dim
    per_w = math.ceil(blocks / NUM_WORKERS)
    mesh = plsc.VectorSubcoreMesh(
        core_axis_name="c", subcore_axis_name="s",
        num_cores=NUM_CORES, num_subcores=NUM_SUBCORES)

    @functools.partial(
        pl.kernel,
        out_type=jax.ShapeDtypeStruct((out_rows, 128), jnp.float32),
        mesh=mesh,
        scratch_types=[
            pltpu.VMEM((2, dim, CHUNK), jnp.float32),
            pltpu.VMEM((2, dim, CHUNK), jnp.float32),
            pltpu.VMEM((tail_out, 128), jnp.float32),
            pltpu.SemaphoreType.DMA,
            pltpu.SemaphoreType.DMA,
            pltpu.SemaphoreType.DMA,
            pltpu.SemaphoreType.DMA,
        ],
        compiler_params=pltpu.CompilerParams(
            use_tc_tiling_on_sc=True, needs_layout_passes=False),
    )
    def tkern(tab_t_hbm, tail_hbm, out_hbm, inb, outb, tailv,
              isem0, isem1, osem0, osem1):
        wid = lax.axis_index("s") * NUM_CORES + lax.axis_index("c")
        start = wid * blocks // NUM_WORKERS
        isems = (isem0, isem1)
        osems = (osem0, osem1)
        iota = lax.iota(jnp.int32, LANES)

        def in_sl(j):
            off = pl.multiple_of((start + j) * CHUNK, CHUNK)
            return tab_t_hbm.at[:, pl.ds(off, CHUNK)]

        def out_sl(j):
            off = pl.multiple_of((start + j) * dim, dim)
            return out_hbm.at[pl.ds(off, dim)]

        def fire_in(j, p):
            pltpu.async_copy(in_sl(j), inb.at[p], isems[p])

        def wait_in(j, p):
            pltpu.make_async_copy(in_sl(j), inb.at[p], isems[p]).wait()

        def fire_out(j, p):
            pltpu.async_copy(outb.at[p], out_sl(j), osems[p])

        def wait_out(j, p):
            pltpu.make_async_copy(outb.at[p], out_sl(j), osems[p]).wait()

        def transpose_block(p):
            @plsc.parallel_loop(0, CHUNK // 8, unroll=4)
            def rbody(rr):
                for k in range(8):
                    r = rr * 8 + k
                    va = plsc.load_gather(inb.at[p], [iota, iota * 0 + r])
                    outb[p, rr, pl.ds(k * dim, dim)] = va

        def step(j, p, prefetch_guard):
            wait_in(j, p)
            transpose_block(p)
            fire_out(j, p)
            if prefetch_guard is None:
                fire_in(j + 2, p)
            elif prefetch_guard:
                @pl.when(j + 2 < per_w)
                def _():
                    fire_in(j + 2, p)

        fire_in(0, 0)
        fire_in(1, 1)
        step(0, 0, None)
        step(1, 1, None)

        def jbody(jj, _):
            for sub in range(2):
                j = jj * 2 + sub
                p = sub
                wait_in(j, p)
                wait_out(j - 2, p)
                transpose_block(p)
                fire_out(j, p)

                @pl.when(j + 2 < per_w)
                def _():
                    fire_in(j + 2, p)
            return 0

        half = per_w // 2
        lax.fori_loop(1, half, jbody, 0)
        j_last = per_w - 1
        wait_in(j_last, 0)
        wait_out(j_last - 2, 0)
        transpose_block(0)
        fire_out(j_last, 0)
        wait_out(j_last - 1, 1)
        wait_out(j_last, 0)

        @pl.when(wid == 0)
        def _():
            pltpu.sync_copy(tail_hbm, tailv)
            pltpu.sync_copy(tailv, out_hbm.at[pl.ds(blocks * dim, tail_out)])

    return tkern, blocks


@functools.cache
def _build_gather(batch, table_slices, dim):
    lp = 128 // dim
    lp_shift = lp.bit_length() - 1
    b_per_w = batch // NUM_WORKERS
    n_chunks = b_per_w // CHUNK
    gpc = CHUNK // LANES
    out_rows_w = b_per_w * dim // 128
    mesh = plsc.VectorSubcoreMesh(
        core_axis_name="c", subcore_axis_name="s",
        num_cores=NUM_CORES, num_subcores=NUM_SUBCORES)

    @functools.partial(
        pl.kernel,
        out_type=jax.ShapeDtypeStruct((batch * dim // 128, 128), jnp.float32),
        mesh=mesh,
        scratch_types=[
            pltpu.VMEM((b_per_w,), jnp.int32),
            pltpu.VMEM((b_per_w,), jnp.int32),
            pltpu.VMEM((n_chunks, CHUNK), jnp.int32),
            pltpu.VMEM((n_chunks, CHUNK), jnp.int32),
            pltpu.VMEM((2, CHUNK, 128), jnp.float32),
            pltpu.VMEM((2, CHUNK, 128), jnp.float32),
            pltpu.VMEM((out_rows_w, 128), jnp.float32),
            pltpu.SemaphoreType.DMA,
            pltpu.SemaphoreType.DMA,
        ],
        compiler_params=pltpu.CompilerParams(
            use_tc_tiling_on_sc=True, needs_layout_passes=False),
    )
    def gmf(idx_a_hbm, idx_b_hbm, table_hbm, out_hbm,
            idx_av, idx_bv, qa_v, qb_v, buf_a, buf_b, out_v, sem0, sem1):
        wid = lax.axis_index("s") * NUM_CORES + lax.axis_index("c")
        base = wid * b_per_w
        pltpu.sync_copy(idx_a_hbm.at[pl.ds(base, b_per_w)], idx_av)
        pltpu.sync_copy(idx_b_hbm.at[pl.ds(base, b_per_w)], idx_bv)

        for k in range(b_per_w // LANES):
            sl = pl.ds(k * LANES, LANES)
            dst = pl.ds((k % gpc) * LANES, LANES)
            qa_v[k // gpc, dst] = lax.shift_right_logical(idx_av[sl], lp_shift)
            qb_v[k // gpc, dst] = lax.shift_right_logical(idx_bv[sl], lp_shift)

        sems = (sem0, sem1)

        def fire(j):
            p = j % 2
            return (
                pltpu.async_copy(table_hbm.at[qa_v.at[j]], buf_a.at[p], sems[p]),
                pltpu.async_copy(table_hbm.at[qb_v.at[j]], buf_b.at[p], sems[p]),
            )

        iota = lax.iota(jnp.int32, LANES)
        ocol = jnp.bitwise_and(iota, lp - 1) * dim
        orow_l = lax.shift_right_logical(iota, lp_shift)

        pending = {0: fire(0)}
        if n_chunks > 1:
            pending[1] = fire(1)
        for j in range(n_chunks):
            p = j % 2
            da, db = pending.pop(j)
            da.wait()
            db.wait()
            for g in range(gpc):
                rows_i = iota + g * LANES
                sl = pl.ds(j * CHUNK + g * LANES, LANES)
                cola = jnp.bitwise_and(idx_av[sl], lp - 1) * dim
                colb = jnp.bitwise_and(idx_bv[sl], lp - 1) * dim
                orow = orow_l + (j * CHUNK + g * LANES) // lp

                def dbody(d, _, rows_i=rows_i, cola=cola, colb=colb,
                          orow=orow, p=p):
                    va = plsc.load_gather(buf_a.at[p], [rows_i, cola + d])
                    vb = plsc.load_gather(buf_b.at[p], [rows_i, colb + d])
                    plsc.store_scatter(out_v, [orow, ocol + d], va * vb)
                    return 0

                lax.fori_loop(0, dim, dbody, 0, unroll=4)
            if j + 2 < n_chunks:
                pending[j + 2] = fire(j + 2)

        pltpu.sync_copy(out_v, out_hbm.at[pl.ds(wid * out_rows_w, out_rows_w)])

    return gmf


def kernel(input_plylst, input_item, table_plylst, table_item):
    batch = input_plylst.shape[0]
    n_rows, dim = table_plylst.shape
    idx_a = input_plylst.astype(jnp.int32)
    idx_b = input_item.astype(jnp.int32)
    tkern, blocks = _build_transpose(n_rows, dim)
    tail = table_plylst[blocks * CHUNK:].reshape(-1, 128)
    table128 = tkern(table_plylst.T, tail)
    out = _build_gather(batch, table128.shape[0], dim)(idx_a, idx_b, table128)
    return out.reshape(batch, dim)


# speedup vs baseline: 2.4429x; 1.4825x over previous
import functools
import math

import jax
import jax.numpy as jnp
from jax import lax
from jax.experimental import pallas as pl
from jax.experimental.pallas import tpu as pltpu
from jax.experimental.pallas import tpu_sc as plsc

NUM_CORES = 2
NUM_SUBCORES = 16
NUM_WORKERS = NUM_CORES * NUM_SUBCORES
LANES = 16
CHUNK = 128


@functools.cache
def _build_transpose(n_rows, dim):
    blocks = n_rows // CHUNK
    out_rows = n_rows * dim // 128
    tail_out = out_rows - blocks * dim
    per_w = math.ceil(blocks / NUM_WORKERS)
    mesh = plsc.VectorSubcoreMesh(
        core_axis_name="c", subcore_axis_name="s",
        num_cores=NUM_CORES, num_subcores=NUM_SUBCORES)

    @functools.partial(
        pl.kernel,
        out_type=jax.ShapeDtypeStruct((out_rows, 128), jnp.float32),
        mesh=mesh,
        scratch_types=[
            pltpu.VMEM((2, dim, CHUNK), jnp.float32),
            pltpu.VMEM((2, dim, CHUNK), jnp.float32),
            pltpu.VMEM((tail_out, 128), jnp.float32),
            pltpu.SemaphoreType.DMA,
            pltpu.SemaphoreType.DMA,
            pltpu.SemaphoreType.DMA,
            pltpu.SemaphoreType.DMA,
        ],
        compiler_params=pltpu.CompilerParams(
            use_tc_tiling_on_sc=True, needs_layout_passes=False),
    )
    def tkern(tab_t_hbm, tail_hbm, out_hbm, inb, outb, tailv,
              isem0, isem1, osem0, osem1):
        wid = lax.axis_index("s") * NUM_CORES + lax.axis_index("c")
        start = wid * blocks // NUM_WORKERS
        isems = (isem0, isem1)
        osems = (osem0, osem1)
        iota = lax.iota(jnp.int32, LANES)

        def in_sl(j):
            off = pl.multiple_of((start + j) * CHUNK, CHUNK)
            return tab_t_hbm.at[:, pl.ds(off, CHUNK)]

        def out_sl(j):
            off = pl.multiple_of((start + j) * dim, dim)
            return out_hbm.at[pl.ds(off, dim)]

        def fire_in(j, p):
            pltpu.async_copy(in_sl(j), inb.at[p], isems[p])

        def wait_in(j, p):
            pltpu.make_async_copy(in_sl(j), inb.at[p], isems[p]).wait()

        def fire_out(j, p):
            pltpu.async_copy(outb.at[p], out_sl(j), osems[p])

        def wait_out(j, p):
            pltpu.make_async_copy(outb.at[p], out_sl(j), osems[p]).wait()

        rowv = tuple(2 * g + lax.shift_right_logical(iota, 3)
                     for g in range(CHUNK // LANES))
        colbase = jnp.bitwise_and(iota, 7) * dim

        def transpose_block(p):
            @plsc.parallel_loop(0, dim, unroll=2)
            def dbody(d):
                colv = colbase + d
                for g in range(CHUNK // LANES):
                    v = inb[p, d, pl.ds(g * LANES, LANES)]
                    plsc.store_scatter(outb.at[p], [rowv[g], colv], v)

        def step(j, p, prefetch_guard):
            wait_in(j, p)
            transpose_block(p)
            fire_out(j, p)
            if prefetch_guard is None:
                fire_in(j + 2, p)
            elif prefetch_guard:
                @pl.when(j + 2 < per_w)
                def _():
                    fire_in(j + 2, p)

        fire_in(0, 0)
        fire_in(1, 1)
        step(0, 0, None)
        step(1, 1, None)

        def jbody(jj, _):
            for sub in range(2):
                j = jj * 2 + sub
                p = sub
                wait_in(j, p)
                wait_out(j - 2, p)
                transpose_block(p)
                fire_out(j, p)

                @pl.when(j + 2 < per_w)
                def _():
                    fire_in(j + 2, p)
            return 0

        half = per_w // 2
        lax.fori_loop(1, half, jbody, 0)
        j_last = per_w - 1
        wait_in(j_last, 0)
        wait_out(j_last - 2, 0)
        transpose_block(0)
        fire_out(j_last, 0)
        wait_out(j_last - 1, 1)
        wait_out(j_last, 0)

        @pl.when(wid == 0)
        def _():
            pltpu.sync_copy(tail_hbm, tailv)
            pltpu.sync_copy(tailv, out_hbm.at[pl.ds(blocks * dim, tail_out)])

    return tkern, blocks


@functools.cache
def _build_gather(batch, table_slices, dim):
    lp = 128 // dim
    lp_shift = lp.bit_length() - 1
    b_per_w = batch // NUM_WORKERS
    n_chunks = b_per_w // CHUNK
    gpc = CHUNK // LANES
    out_rows_w = b_per_w * dim // 128
    mesh = plsc.VectorSubcoreMesh(
        core_axis_name="c", subcore_axis_name="s",
        num_cores=NUM_CORES, num_subcores=NUM_SUBCORES)

    @functools.partial(
        pl.kernel,
        out_type=jax.ShapeDtypeStruct((batch * dim // 128, 128), jnp.float32),
        mesh=mesh,
        scratch_types=[
            pltpu.VMEM((b_per_w,), jnp.int32),
            pltpu.VMEM((b_per_w,), jnp.int32),
            pltpu.VMEM((n_chunks, CHUNK), jnp.int32),
            pltpu.VMEM((n_chunks, CHUNK), jnp.int32),
            pltpu.VMEM((2, CHUNK, 128), jnp.float32),
            pltpu.VMEM((2, CHUNK, 128), jnp.float32),
            pltpu.VMEM((out_rows_w, 128), jnp.float32),
            pltpu.SemaphoreType.DMA,
            pltpu.SemaphoreType.DMA,
        ],
        compiler_params=pltpu.CompilerParams(
            use_tc_tiling_on_sc=True, needs_layout_passes=False),
    )
    def gmf(idx_a_hbm, idx_b_hbm, table_hbm, out_hbm,
            idx_av, idx_bv, qa_v, qb_v, buf_a, buf_b, out_v, sem0, sem1):
        wid = lax.axis_index("s") * NUM_CORES + lax.axis_index("c")
        base = wid * b_per_w
        pltpu.sync_copy(idx_a_hbm.at[pl.ds(base, b_per_w)], idx_av)
        pltpu.sync_copy(idx_b_hbm.at[pl.ds(base, b_per_w)], idx_bv)

        for k in range(b_per_w // LANES):
            sl = pl.ds(k * LANES, LANES)
            dst = pl.ds((k % gpc) * LANES, LANES)
            qa_v[k // gpc, dst] = lax.shift_right_logical(idx_av[sl], lp_shift)
            qb_v[k // gpc, dst] = lax.shift_right_logical(idx_bv[sl], lp_shift)

        sems = (sem0, sem1)

        def fire(j):
            p = j % 2
            return (
                pltpu.async_copy(table_hbm.at[qa_v.at[j]], buf_a.at[p], sems[p]),
                pltpu.async_copy(table_hbm.at[qb_v.at[j]], buf_b.at[p], sems[p]),
            )

        iota = lax.iota(jnp.int32, LANES)
        ocol = jnp.bitwise_and(iota, lp - 1) * dim
        orow_l = lax.shift_right_logical(iota, lp_shift)

        pending = {0: fire(0)}
        if n_chunks > 1:
            pending[1] = fire(1)
        for j in range(n_chunks):
            p = j % 2
            da, db = pending.pop(j)
            da.wait()
            db.wait()
            for g in range(gpc):
                rows_i = iota + g * LANES
                sl = pl.ds(j * CHUNK + g * LANES, LANES)
                cola = jnp.bitwise_and(idx_av[sl], lp - 1) * dim
                colb = jnp.bitwise_and(idx_bv[sl], lp - 1) * dim
                orow = orow_l + (j * CHUNK + g * LANES) // lp

                def dbody(d, _, rows_i=rows_i, cola=cola, colb=colb,
                          orow=orow, p=p):
                    va = plsc.load_gather(buf_a.at[p], [rows_i, cola + d])
                    vb = plsc.load_gather(buf_b.at[p], [rows_i, colb + d])
                    plsc.store_scatter(out_v, [orow, ocol + d], va * vb)
                    return 0

                lax.fori_loop(0, dim, dbody, 0, unroll=4)
            if j + 2 < n_chunks:
                pending[j + 2] = fire(j + 2)

        pltpu.sync_copy(out_v, out_hbm.at[pl.ds(wid * out_rows_w, out_rows_w)])

    return gmf


def kernel(input_plylst, input_item, table_plylst, table_item):
    batch = input_plylst.shape[0]
    n_rows, dim = table_plylst.shape
    idx_a = input_plylst.astype(jnp.int32)
    idx_b = input_item.astype(jnp.int32)
    tkern, blocks = _build_transpose(n_rows, dim)
    tail = table_plylst[blocks * CHUNK:].reshape(-1, 128)
    table128 = tkern(table_plylst.T, tail)
    out = _build_gather(batch, table128.shape[0], dim)(idx_a, idx_b, table128)
    return out.reshape(batch, dim)


# speedup vs baseline: 2.4694x; 1.0109x over previous
import functools
import math

import jax
import jax.numpy as jnp
from jax import lax
from jax.experimental import pallas as pl
from jax.experimental.pallas import tpu as pltpu
from jax.experimental.pallas import tpu_sc as plsc

NUM_CORES = 2
NUM_SUBCORES = 16
NUM_WORKERS = NUM_CORES * NUM_SUBCORES
LANES = 16
CHUNK = 128


@functools.cache
def _build_transpose(n_rows, dim):
    blocks = n_rows // CHUNK
    out_rows = n_rows * dim // 128
    tail_out = out_rows - blocks * dim
    per_w = math.ceil(blocks / NUM_WORKERS)
    mesh = plsc.VectorSubcoreMesh(
        core_axis_name="c", subcore_axis_name="s",
        num_cores=NUM_CORES, num_subcores=NUM_SUBCORES)

    @functools.partial(
        pl.kernel,
        out_type=jax.ShapeDtypeStruct((out_rows, 128), jnp.float32),
        mesh=mesh,
        scratch_types=[
            pltpu.VMEM((2, dim, CHUNK), jnp.float32),
            pltpu.VMEM((2, dim, CHUNK), jnp.float32),
            pltpu.VMEM((tail_out, 128), jnp.float32),
            pltpu.SemaphoreType.DMA,
            pltpu.SemaphoreType.DMA,
            pltpu.SemaphoreType.DMA,
            pltpu.SemaphoreType.DMA,
        ],
        compiler_params=pltpu.CompilerParams(
            use_tc_tiling_on_sc=True, needs_layout_passes=False),
    )
    def tkern(tab_t_hbm, tail_hbm, out_hbm, inb, outb, tailv,
              isem0, isem1, osem0, osem1):
        wid = lax.axis_index("s") * NUM_CORES + lax.axis_index("c")
        start = wid * blocks // NUM_WORKERS
        isems = (isem0, isem1)
        osems = (osem0, osem1)
        iota = lax.iota(jnp.int32, LANES)

        def in_sl(j):
            off = pl.multiple_of((start + j) * CHUNK, CHUNK)
            return tab_t_hbm.at[:, pl.ds(off, CHUNK)]

        def out_sl(j):
            off = pl.multiple_of((start + j) * dim, dim)
            return out_hbm.at[pl.ds(off, dim)]

        def fire_in(j, p):
            pltpu.async_copy(in_sl(j), inb.at[p], isems[p])

        def wait_in(j, p):
            pltpu.make_async_copy(in_sl(j), inb.at[p], isems[p]).wait()

        def fire_out(j, p):
            pltpu.async_copy(outb.at[p], out_sl(j), osems[p])

        def wait_out(j, p):
            pltpu.make_async_copy(outb.at[p], out_sl(j), osems[p]).wait()

        rowv = tuple(2 * g + lax.shift_right_logical(iota, 3)
                     for g in range(CHUNK // LANES))
        colbase = jnp.bitwise_and(iota, 7) * dim

        def transpose_block(p):
            @plsc.parallel_loop(0, dim, unroll=4)
            def dbody(d):
                colv = colbase + d
                for g in range(CHUNK // LANES):
                    v = inb[p, d, pl.ds(g * LANES, LANES)]
                    plsc.store_scatter(outb.at[p], [rowv[g], colv], v)

        def step(j, p, prefetch_guard):
            wait_in(j, p)
            transpose_block(p)
            fire_out(j, p)
            if prefetch_guard is None:
                fire_in(j + 2, p)
            elif prefetch_guard:
                @pl.when(j + 2 < per_w)
                def _():
                    fire_in(j + 2, p)

        fire_in(0, 0)
        fire_in(1, 1)
        step(0, 0, None)
        step(1, 1, None)

        def jbody(jj, _):
            for sub in range(2):
                j = jj * 2 + sub
                p = sub
                wait_in(j, p)
                wait_out(j - 2, p)
                transpose_block(p)
                fire_out(j, p)

                @pl.when(j + 2 < per_w)
                def _():
                    fire_in(j + 2, p)
            return 0

        half = per_w // 2
        lax.fori_loop(1, half, jbody, 0)
        j_last = per_w - 1
        wait_in(j_last, 0)
        wait_out(j_last - 2, 0)
        transpose_block(0)
        fire_out(j_last, 0)
        wait_out(j_last - 1, 1)
        wait_out(j_last, 0)

        @pl.when(wid == 0)
        def _():
            pltpu.sync_copy(tail_hbm, tailv)
            pltpu.sync_copy(tailv, out_hbm.at[pl.ds(blocks * dim, tail_out)])

    return tkern, blocks


@functools.cache
def _build_gather(batch, table_slices, dim):
    lp = 128 // dim
    lp_shift = lp.bit_length() - 1
    b_per_w = batch // NUM_WORKERS
    n_chunks = b_per_w // CHUNK
    gpc = CHUNK // LANES
    out_rows_w = b_per_w * dim // 128
    mesh = plsc.VectorSubcoreMesh(
        core_axis_name="c", subcore_axis_name="s",
        num_cores=NUM_CORES, num_subcores=NUM_SUBCORES)

    @functools.partial(
        pl.kernel,
        out_type=jax.ShapeDtypeStruct((batch * dim // 128, 128), jnp.float32),
        mesh=mesh,
        scratch_types=[
            pltpu.VMEM((b_per_w,), jnp.int32),
            pltpu.VMEM((b_per_w,), jnp.int32),
            pltpu.VMEM((n_chunks, CHUNK), jnp.int32),
            pltpu.VMEM((n_chunks, CHUNK), jnp.int32),
            pltpu.VMEM((2, CHUNK, 128), jnp.float32),
            pltpu.VMEM((2, CHUNK, 128), jnp.float32),
            pltpu.VMEM((out_rows_w, 128), jnp.float32),
            pltpu.SemaphoreType.DMA,
            pltpu.SemaphoreType.DMA,
        ],
        compiler_params=pltpu.CompilerParams(
            use_tc_tiling_on_sc=True, needs_layout_passes=False),
    )
    def gmf(idx_a_hbm, idx_b_hbm, table_hbm, out_hbm,
            idx_av, idx_bv, qa_v, qb_v, buf_a, buf_b, out_v, sem0, sem1):
        wid = lax.axis_index("s") * NUM_CORES + lax.axis_index("c")
        base = wid * b_per_w
        pltpu.sync_copy(idx_a_hbm.at[pl.ds(base, b_per_w)], idx_av)
        pltpu.sync_copy(idx_b_hbm.at[pl.ds(base, b_per_w)], idx_bv)

        for k in range(b_per_w // LANES):
            sl = pl.ds(k * LANES, LANES)
            dst = pl.ds((k % gpc) * LANES, LANES)
            qa_v[k // gpc, dst] = lax.shift_right_logical(idx_av[sl], lp_shift)
            qb_v[k // gpc, dst] = lax.shift_right_logical(idx_bv[sl], lp_shift)

        sems = (sem0, sem1)

        def fire(j):
            p = j % 2
            return (
                pltpu.async_copy(table_hbm.at[qa_v.at[j]], buf_a.at[p], sems[p]),
                pltpu.async_copy(table_hbm.at[qb_v.at[j]], buf_b.at[p], sems[p]),
            )

        iota = lax.iota(jnp.int32, LANES)
        ocol = jnp.bitwise_and(iota, lp - 1) * dim
        orow_l = lax.shift_right_logical(iota, lp_shift)

        pending = {0: fire(0)}
        if n_chunks > 1:
            pending[1] = fire(1)
        for j in range(n_chunks):
            p = j % 2
            da, db = pending.pop(j)
            da.wait()
            db.wait()
            for g in range(gpc):
                rows_i = iota + g * LANES
                sl = pl.ds(j * CHUNK + g * LANES, LANES)
                cola = jnp.bitwise_and(idx_av[sl], lp - 1) * dim
                colb = jnp.bitwise_and(idx_bv[sl], lp - 1) * dim
                orow = orow_l + (j * CHUNK + g * LANES) // lp

                def dbody(d, _, rows_i=rows_i, cola=cola, colb=colb,
                          orow=orow, p=p):
                    va = plsc.load_gather(buf_a.at[p], [rows_i, cola + d])
                    vb = plsc.load_gather(buf_b.at[p], [rows_i, colb + d])
                    plsc.store_scatter(out_v, [orow, ocol + d], va * vb)
                    return 0

                lax.fori_loop(0, dim, dbody, 0, unroll=4)
            if j + 2 < n_chunks:
                pending[j + 2] = fire(j + 2)

        pltpu.sync_copy(out_v, out_hbm.at[pl.ds(wid * out_rows_w, out_rows_w)])

    return gmf


def kernel(input_plylst, input_item, table_plylst, table_item):
    batch = input_plylst.shape[0]
    n_rows, dim = table_plylst.shape
    idx_a = input_plylst.astype(jnp.int32)
    idx_b = input_item.astype(jnp.int32)
    tkern, blocks = _build_transpose(n_rows, dim)
    tail = table_plylst[blocks * CHUNK:].reshape(-1, 128)
    table128 = tkern(table_plylst.T, tail)
    out = _build_gather(batch, table128.shape[0], dim)(idx_a, idx_b, table128)
    return out.reshape(batch, dim)


# speedup vs baseline: 3.4637x; 1.4027x over previous
import functools
import math

import jax
import jax.numpy as jnp
from jax import lax
from jax.experimental import pallas as pl
from jax.experimental.pallas import tpu as pltpu
from jax.experimental.pallas import tpu_sc as plsc

NUM_CORES = 2
NUM_SUBCORES = 16
NUM_WORKERS = NUM_CORES * NUM_SUBCORES
LANES = 16
CHUNK = 128


@functools.cache
def _build_transpose(n_rows, dim):
    blocks = n_rows // CHUNK
    out_rows = n_rows * dim // 128
    tail_out = out_rows - blocks * dim
    per_w = math.ceil(blocks / NUM_WORKERS)
    mesh = plsc.VectorSubcoreMesh(
        core_axis_name="c", subcore_axis_name="s",
        num_cores=NUM_CORES, num_subcores=NUM_SUBCORES)

    @functools.partial(
        pl.kernel,
        out_type=jax.ShapeDtypeStruct((out_rows, 128), jnp.float32),
        mesh=mesh,
        scratch_types=[
            pltpu.VMEM((4, dim, CHUNK), jnp.float32),
            pltpu.VMEM((4, dim, CHUNK), jnp.float32),
            pltpu.VMEM((tail_out, 128), jnp.float32),
            pltpu.SemaphoreType.DMA,
            pltpu.SemaphoreType.DMA,
            pltpu.SemaphoreType.DMA,
            pltpu.SemaphoreType.DMA,
            pltpu.SemaphoreType.DMA,
            pltpu.SemaphoreType.DMA,
            pltpu.SemaphoreType.DMA,
            pltpu.SemaphoreType.DMA,
        ],
        compiler_params=pltpu.CompilerParams(
            use_tc_tiling_on_sc=True, needs_layout_passes=False),
    )
    def tkern(tab_t_hbm, tail_hbm, out_hbm, inb, outb, tailv,
              isem0, isem1, isem2, isem3, osem0, osem1, osem2, osem3):
        wid = lax.axis_index("s") * NUM_CORES + lax.axis_index("c")
        start = wid * blocks // NUM_WORKERS
        isems = (isem0, isem1, isem2, isem3)
        osems = (osem0, osem1, osem2, osem3)
        iota = lax.iota(jnp.int32, LANES)

        def in_sl(j):
            off = pl.multiple_of((start + j) * CHUNK, CHUNK)
            return tab_t_hbm.at[:, pl.ds(off, CHUNK)]

        def out_sl(j):
            off = pl.multiple_of((start + j) * dim, dim)
            return out_hbm.at[pl.ds(off, dim)]

        def fire_in(j, p):
            pltpu.async_copy(in_sl(j), inb.at[p], isems[p])

        def wait_in(j, p):
            pltpu.make_async_copy(in_sl(j), inb.at[p], isems[p]).wait()

        def fire_out(j, p):
            pltpu.async_copy(outb.at[p], out_sl(j), osems[p])

        def wait_out(j, p):
            pltpu.make_async_copy(outb.at[p], out_sl(j), osems[p]).wait()

        rowv = tuple(2 * g + lax.shift_right_logical(iota, 3)
                     for g in range(CHUNK // LANES))
        colbase = jnp.bitwise_and(iota, 7) * dim

        def transpose_block(p):
            @plsc.parallel_loop(0, dim, unroll=4)
            def dbody(d):
                colv = colbase + d
                for g in range(CHUNK // LANES):
                    v = inb[p, d, pl.ds(g * LANES, LANES)]
                    plsc.store_scatter(outb.at[p], [rowv[g], colv], v)

        RING = 4
        for j in range(RING):
            fire_in(j, j)
        for j in range(RING):
            wait_in(j, j)
            transpose_block(j)
            fire_out(j, j)
            fire_in(j + RING, j)

        def jbody(jj, _):
            for sub in range(RING):
                j = jj * RING + sub
                p = sub
                wait_in(j, p)
                wait_out(j - RING, p)
                transpose_block(p)
                fire_out(j, p)

                @pl.when(j + RING < per_w)
                def _():
                    fire_in(j + RING, p)
            return 0

        nfull = per_w // RING
        lax.fori_loop(1, nfull, jbody, 0)
        for j in range(RING * nfull, per_w):
            p = j % RING
            wait_in(j, p)
            wait_out(j - RING, p)
            transpose_block(p)
            fire_out(j, p)
        for j in range(per_w - RING, per_w):
            wait_out(j, j % RING)

        @pl.when(wid == 0)
        def _():
            pltpu.sync_copy(tail_hbm, tailv)
            pltpu.sync_copy(tailv, out_hbm.at[pl.ds(blocks * dim, tail_out)])

    return tkern, blocks


@functools.cache
def _build_gather(batch, table_slices, dim):
    lp = 128 // dim
    lp_shift = lp.bit_length() - 1
    b_per_w = batch // NUM_WORKERS
    n_chunks = b_per_w // CHUNK
    gpc = CHUNK // LANES
    out_rows_w = b_per_w * dim // 128
    mesh = plsc.VectorSubcoreMesh(
        core_axis_name="c", subcore_axis_name="s",
        num_cores=NUM_CORES, num_subcores=NUM_SUBCORES)

    @functools.partial(
        pl.kernel,
        out_type=jax.ShapeDtypeStruct((batch * dim // 128, 128), jnp.float32),
        mesh=mesh,
        scratch_types=[
            pltpu.VMEM((b_per_w,), jnp.int32),
            pltpu.VMEM((b_per_w,), jnp.int32),
            pltpu.VMEM((n_chunks, CHUNK), jnp.int32),
            pltpu.VMEM((n_chunks, CHUNK), jnp.int32),
            pltpu.VMEM((2, CHUNK, 128), jnp.float32),
            pltpu.VMEM((2, CHUNK, 128), jnp.float32),
            pltpu.VMEM((out_rows_w, 128), jnp.float32),
            pltpu.SemaphoreType.DMA,
            pltpu.SemaphoreType.DMA,
        ],
        compiler_params=pltpu.CompilerParams(
            use_tc_tiling_on_sc=True, needs_layout_passes=False),
    )
    def gmf(idx_a_hbm, idx_b_hbm, table_hbm, out_hbm,
            idx_av, idx_bv, qa_v, qb_v, buf_a, buf_b, out_v, sem0, sem1):
        wid = lax.axis_index("s") * NUM_CORES + lax.axis_index("c")
        base = wid * b_per_w
        pltpu.sync_copy(idx_a_hbm.at[pl.ds(base, b_per_w)], idx_av)
        pltpu.sync_copy(idx_b_hbm.at[pl.ds(base, b_per_w)], idx_bv)

        for k in range(b_per_w // LANES):
            sl = pl.ds(k * LANES, LANES)
            dst = pl.ds((k % gpc) * LANES, LANES)
            qa_v[k // gpc, dst] = lax.shift_right_logical(idx_av[sl], lp_shift)
            qb_v[k // gpc, dst] = lax.shift_right_logical(idx_bv[sl], lp_shift)

        sems = (sem0, sem1)

        def fire(j):
            p = j % 2
            return (
                pltpu.async_copy(table_hbm.at[qa_v.at[j]], buf_a.at[p], sems[p]),
                pltpu.async_copy(table_hbm.at[qb_v.at[j]], buf_b.at[p], sems[p]),
            )

        iota = lax.iota(jnp.int32, LANES)
        ocol = jnp.bitwise_and(iota, lp - 1) * dim
        orow_l = lax.shift_right_logical(iota, lp_shift)

        pending = {0: fire(0)}
        if n_chunks > 1:
            pending[1] = fire(1)
        for j in range(n_chunks):
            p = j % 2
            da, db = pending.pop(j)
            da.wait()
            db.wait()
            for g in range(gpc):
                rows_i = iota + g * LANES
                sl = pl.ds(j * CHUNK + g * LANES, LANES)
                cola = jnp.bitwise_and(idx_av[sl], lp - 1) * dim
                colb = jnp.bitwise_and(idx_bv[sl], lp - 1) * dim
                orow = orow_l + (j * CHUNK + g * LANES) // lp

                def dbody(d, _, rows_i=rows_i, cola=cola, colb=colb,
                          orow=orow, p=p):
                    va = plsc.load_gather(buf_a.at[p], [rows_i, cola + d])
                    vb = plsc.load_gather(buf_b.at[p], [rows_i, colb + d])
                    plsc.store_scatter(out_v, [orow, ocol + d], va * vb)
                    return 0

                lax.fori_loop(0, dim, dbody, 0, unroll=4)
            if j + 2 < n_chunks:
                pending[j + 2] = fire(j + 2)

        pltpu.sync_copy(out_v, out_hbm.at[pl.ds(wid * out_rows_w, out_rows_w)])

    return gmf


def kernel(input_plylst, input_item, table_plylst, table_item):
    batch = input_plylst.shape[0]
    n_rows, dim = table_plylst.shape
    idx_a = input_plylst.astype(jnp.int32)
    idx_b = input_item.astype(jnp.int32)
    tkern, blocks = _build_transpose(n_rows, dim)
    tail = table_plylst[blocks * CHUNK:].reshape(-1, 128)
    table128 = tkern(table_plylst.T, tail)
    out = _build_gather(batch, table128.shape[0], dim)(idx_a, idx_b, table128)
    return out.reshape(batch, dim)


# speedup vs baseline: 4.2089x; 1.2151x over previous
import functools
import math

import jax
import jax.numpy as jnp
from jax import lax
from jax.experimental import pallas as pl
from jax.experimental.pallas import tpu as pltpu
from jax.experimental.pallas import tpu_sc as plsc

NUM_CORES = 2
NUM_SUBCORES = 16
NUM_WORKERS = NUM_CORES * NUM_SUBCORES
LANES = 16
CHUNK = 128


@functools.cache
def _build_transpose(n_rows, dim):
    blocks = n_rows // CHUNK
    out_rows = n_rows * dim // 128
    tail_out = out_rows - blocks * dim
    per_w = math.ceil(blocks / NUM_WORKERS)
    mesh = plsc.VectorSubcoreMesh(
        core_axis_name="c", subcore_axis_name="s",
        num_cores=NUM_CORES, num_subcores=NUM_SUBCORES)

    @functools.partial(
        pl.kernel,
        out_type=jax.ShapeDtypeStruct((out_rows, 128), jnp.float32),
        mesh=mesh,
        scratch_types=[
            pltpu.VMEM((8, dim, CHUNK), jnp.float32),
            pltpu.VMEM((8, dim, CHUNK), jnp.float32),
            pltpu.VMEM((tail_out, 128), jnp.float32),
            pltpu.SemaphoreType.DMA,
            pltpu.SemaphoreType.DMA,
            pltpu.SemaphoreType.DMA,
            pltpu.SemaphoreType.DMA,
            pltpu.SemaphoreType.DMA,
            pltpu.SemaphoreType.DMA,
            pltpu.SemaphoreType.DMA,
            pltpu.SemaphoreType.DMA,
            pltpu.SemaphoreType.DMA,
            pltpu.SemaphoreType.DMA,
            pltpu.SemaphoreType.DMA,
            pltpu.SemaphoreType.DMA,
            pltpu.SemaphoreType.DMA,
            pltpu.SemaphoreType.DMA,
            pltpu.SemaphoreType.DMA,
            pltpu.SemaphoreType.DMA,
        ],
        compiler_params=pltpu.CompilerParams(
            use_tc_tiling_on_sc=True, needs_layout_passes=False),
    )
    def tkern(tab_t_hbm, tail_hbm, out_hbm, inb, outb, tailv, *sems16):
        wid = lax.axis_index("s") * NUM_CORES + lax.axis_index("c")
        start = wid * blocks // NUM_WORKERS
        isems = sems16[:8]
        osems = sems16[8:]
        iota = lax.iota(jnp.int32, LANES)

        def in_sl(j):
            off = pl.multiple_of((start + j) * CHUNK, CHUNK)
            return tab_t_hbm.at[:, pl.ds(off, CHUNK)]

        def out_sl(j):
            off = pl.multiple_of((start + j) * dim, dim)
            return out_hbm.at[pl.ds(off, dim)]

        def fire_in(j, p):
            pltpu.async_copy(in_sl(j), inb.at[p], isems[p])

        def wait_in(j, p):
            pltpu.make_async_copy(in_sl(j), inb.at[p], isems[p]).wait()

        def fire_out(j, p):
            pltpu.async_copy(outb.at[p], out_sl(j), osems[p])

        def wait_out(j, p):
            pltpu.make_async_copy(outb.at[p], out_sl(j), osems[p]).wait()

        rowv = tuple(2 * g + lax.shift_right_logical(iota, 3)
                     for g in range(CHUNK // LANES))
        colbase = jnp.bitwise_and(iota, 7) * dim

        def transpose_block(p):
            @plsc.parallel_loop(0, dim, unroll=4)
            def dbody(d):
                colv = colbase + d
                for g in range(CHUNK // LANES):
                    v = inb[p, d, pl.ds(g * LANES, LANES)]
                    plsc.store_scatter(outb.at[p], [rowv[g], colv], v)

        RING = 8
        for j in range(RING):
            fire_in(j, j)
        for j in range(RING):
            wait_in(j, j)
            transpose_block(j)
            fire_out(j, j)
            fire_in(j + RING, j)

        def jbody(jj, _):
            for sub in range(RING):
                j = jj * RING + sub
                p = sub
                wait_in(j, p)
                wait_out(j - RING, p)
                transpose_block(p)
                fire_out(j, p)

                @pl.when(j + RING < per_w)
                def _():
                    fire_in(j + RING, p)
            return 0

        nfull = per_w // RING
        lax.fori_loop(1, nfull, jbody, 0)
        for j in range(RING * nfull, per_w):
            p = j % RING
            wait_in(j, p)
            wait_out(j - RING, p)
            transpose_block(p)
            fire_out(j, p)
        for j in range(per_w - RING, per_w):
            wait_out(j, j % RING)

        @pl.when(wid == 0)
        def _():
            pltpu.sync_copy(tail_hbm, tailv)
            pltpu.sync_copy(tailv, out_hbm.at[pl.ds(blocks * dim, tail_out)])

    return tkern, blocks


@functools.cache
def _build_gather(batch, table_slices, dim):
    lp = 128 // dim
    lp_shift = lp.bit_length() - 1
    b_per_w = batch // NUM_WORKERS
    n_chunks = b_per_w // CHUNK
    gpc = CHUNK // LANES
    out_rows_w = b_per_w * dim // 128
    mesh = plsc.VectorSubcoreMesh(
        core_axis_name="c", subcore_axis_name="s",
        num_cores=NUM_CORES, num_subcores=NUM_SUBCORES)

    @functools.partial(
        pl.kernel,
        out_type=jax.ShapeDtypeStruct((batch * dim // 128, 128), jnp.float32),
        mesh=mesh,
        scratch_types=[
            pltpu.VMEM((b_per_w,), jnp.int32),
            pltpu.VMEM((b_per_w,), jnp.int32),
            pltpu.VMEM((n_chunks, CHUNK), jnp.int32),
            pltpu.VMEM((n_chunks, CHUNK), jnp.int32),
            pltpu.VMEM((2, CHUNK, 128), jnp.float32),
            pltpu.VMEM((2, CHUNK, 128), jnp.float32),
            pltpu.VMEM((out_rows_w, 128), jnp.float32),
            pltpu.SemaphoreType.DMA,
            pltpu.SemaphoreType.DMA,
        ],
        compiler_params=pltpu.CompilerParams(
            use_tc_tiling_on_sc=True, needs_layout_passes=False),
    )
    def gmf(idx_a_hbm, idx_b_hbm, table_hbm, out_hbm,
            idx_av, idx_bv, qa_v, qb_v, buf_a, buf_b, out_v, sem0, sem1):
        wid = lax.axis_index("s") * NUM_CORES + lax.axis_index("c")
        base = wid * b_per_w
        pltpu.sync_copy(idx_a_hbm.at[pl.ds(base, b_per_w)], idx_av)
        pltpu.sync_copy(idx_b_hbm.at[pl.ds(base, b_per_w)], idx_bv)

        for k in range(b_per_w // LANES):
            sl = pl.ds(k * LANES, LANES)
            dst = pl.ds((k % gpc) * LANES, LANES)
            qa_v[k // gpc, dst] = lax.shift_right_logical(idx_av[sl], lp_shift)
            qb_v[k // gpc, dst] = lax.shift_right_logical(idx_bv[sl], lp_shift)

        sems = (sem0, sem1)

        def fire(j):
            p = j % 2
            return (
                pltpu.async_copy(table_hbm.at[qa_v.at[j]], buf_a.at[p], sems[p]),
                pltpu.async_copy(table_hbm.at[qb_v.at[j]], buf_b.at[p], sems[p]),
            )

        iota = lax.iota(jnp.int32, LANES)
        ocol = jnp.bitwise_and(iota, lp - 1) * dim
        orow_l = lax.shift_right_logical(iota, lp_shift)

        pending = {0: fire(0)}
        if n_chunks > 1:
            pending[1] = fire(1)
        for j in range(n_chunks):
            p = j % 2
            da, db = pending.pop(j)
            da.wait()
            db.wait()
            for g in range(gpc):
                rows_i = iota + g * LANES
                sl = pl.ds(j * CHUNK + g * LANES, LANES)
                cola = jnp.bitwise_and(idx_av[sl], lp - 1) * dim
                colb = jnp.bitwise_and(idx_bv[sl], lp - 1) * dim
                orow = orow_l + (j * CHUNK + g * LANES) // lp

                @plsc.parallel_loop(0, dim, unroll=4)
                def dbody(d, rows_i=rows_i, cola=cola, colb=colb,
                          orow=orow, p=p):
                    va = plsc.load_gather(buf_a.at[p], [rows_i, cola + d])
                    vb = plsc.load_gather(buf_b.at[p], [rows_i, colb + d])
                    plsc.store_scatter(out_v, [orow, ocol + d], va * vb)
            if j + 2 < n_chunks:
                pending[j + 2] = fire(j + 2)

        pltpu.sync_copy(out_v, out_hbm.at[pl.ds(wid * out_rows_w, out_rows_w)])

    return gmf


def kernel(input_plylst, input_item, table_plylst, table_item):
    batch = input_plylst.shape[0]
    n_rows, dim = table_plylst.shape
    idx_a = input_plylst.astype(jnp.int32)
    idx_b = input_item.astype(jnp.int32)
    tkern, blocks = _build_transpose(n_rows, dim)
    tail = table_plylst[blocks * CHUNK:].reshape(-1, 128)
    table128 = tkern(table_plylst.T, tail)
    out = _build_gather(batch, table128.shape[0], dim)(idx_a, idx_b, table128)
    return out.reshape(batch, dim)
